# Initial kernel scaffold; baseline (speedup 1.0000x reference)
import functools

import jax
import jax.numpy as jnp
from jax import lax
from jax.experimental import pallas as pl
from jax.experimental.pallas import tpu as pltpu
from jax.experimental.pallas import tpu_sc as plsc

N = 10000
E = 320000
G = 64
NC = 2
NS = 16
L = 16
NW = NC * NS
EPT = E // NW
CH = 80
NCHUNK = EPT // CH
RPT = N // NS

_f32 = jnp.float32


def _make_segsum(w):
  mesh = plsc.VectorSubcoreMesh(
      core_axis_name="c", subcore_axis_name="s", num_cores=NC, num_subcores=NS)

  @functools.partial(
      pl.kernel,
      out_type=jax.ShapeDtypeStruct((NC, N, w), _f32),
      mesh=mesh,
      scratch_types=[
          pltpu.VMEM((NCHUNK, CH), jnp.int32),
          pltpu.VMEM((NCHUNK, CH), jnp.int32),
          pltpu.VMEM((CH, w), _f32),
          pltpu.VMEM((RPT, w), _f32),
          pltpu.VMEM_SHARED((N, w), _f32),
          pltpu.SemaphoreType.DMA,
      ],
  )
  def segsum(table_hbm, src_hbm, dst_hbm, out_hbm,
             src_v, dst_v, rows_v, zero_v, acc_sh, sem):
    cid = lax.axis_index("c")
    sid = lax.axis_index("s")
    wid = cid * NS + sid

    pltpu.sync_copy(src_hbm.at[wid], src_v)
    pltpu.sync_copy(dst_hbm.at[wid], dst_v)

    def zbody(r, carry):
      for c in range(w // L):
        zero_v[r, pl.ds(c * L, L)] = jnp.zeros((L,), _f32)
      return carry
    lax.fori_loop(0, RPT, zbody, 0)
    pltpu.sync_copy(zero_v, acc_sh.at[pl.ds(sid * RPT, RPT)])
    plsc.subcore_barrier()

    def body(j, carry):
      pltpu.async_copy(table_hbm.at[src_v.at[j]], rows_v, sem).wait()
      pltpu.sync_copy(rows_v, acc_sh.at[dst_v.at[j]], add=True)
      return carry
    lax.fori_loop(0, NCHUNK, body, 0)
    plsc.subcore_barrier()

    pltpu.sync_copy(acc_sh.at[pl.ds(sid * RPT, RPT)],
                    out_hbm.at[cid, pl.ds(sid * RPT, RPT)])

  return segsum


_segsum = {w: _make_segsum(w) for w in (16, 32, 64)}


def _relu(x):
  return jnp.maximum(x, 0.0)


def _tc_pre_body(x_ref, wr_ref, wo_ref, b_ref, hrel_ref, hroot_ref):
  x = x_ref[...]
  hrel_ref[...] = jnp.dot(x, wr_ref[...], preferred_element_type=_f32)
  hroot_ref[...] = jnp.dot(x, wo_ref[...], preferred_element_type=_f32) + b_ref[...]


def _tc_combine1_body(p_ref, hroot_ref, h_ref):
  h_ref[...] = _relu(p_ref[0] + p_ref[1] + hroot_ref[...])


def _tc_combine_body(p_ref, h_ref, wr_ref, wo_ref, b_ref, out_ref):
  agg = p_ref[0] + p_ref[1]
  out_ref[...] = _relu(
      jnp.dot(agg, wr_ref[...], preferred_element_type=_f32)
      + jnp.dot(h_ref[...], wo_ref[...], preferred_element_type=_f32)
      + b_ref[...])


def _tc_final_body(p_ref, h_ref, wr_ref, wo_ref, b_ref,
                   l1w_ref, l1b_ref, batch_ref, l2w_ref, l2b_ref, out_ref):
  agg = p_ref[0] + p_ref[1]
  h4 = _relu(
      jnp.dot(agg, wr_ref[...], preferred_element_type=_f32)
      + jnp.dot(h_ref[...], wo_ref[...], preferred_element_type=_f32)
      + b_ref[...])
  hl = jnp.dot(h4, l1w_ref[...], preferred_element_type=_f32) + l1b_ref[...]
  gid = lax.broadcasted_iota(jnp.int32, (G, N), 0)
  mask = (gid == batch_ref[...]).astype(_f32)
  sums = jnp.dot(mask, hl, preferred_element_type=_f32)
  counts = jnp.sum(mask, axis=1, keepdims=True)
  pooled = sums / jnp.maximum(counts, 1.0)
  out_ref[...] = jnp.dot(pooled, l2w_ref[...], preferred_element_type=_f32) + l2b_ref[...]


def _sds(shape):
  return jax.ShapeDtypeStruct(shape, _f32)


_tc_pre = pl.pallas_call(
    _tc_pre_body, out_shape=(_sds((N, 16)), _sds((N, 16))))

_tc_combine1 = pl.pallas_call(
    _tc_combine1_body, out_shape=_sds((N, 16)))


def _tc_combine(p, h, wr, wo, b):
  dout = wr.shape[1]
  return pl.pallas_call(_tc_combine_body, out_shape=_sds((N, dout)))(
      p, h, wr, wo, b)


_tc_final = pl.pallas_call(_tc_final_body, out_shape=_sds((G, 1)))


def kernel(x, edge_index, batch, W1_rel, W1_root, b1, W2_rel, W2_root, b2,
           W3_rel, W3_root, b3, W4_rel, W4_root, b4, lin1_W, lin1_b,
           lin2_W, lin2_b):
  src = edge_index[0].reshape(NW, NCHUNK, CH)
  dst = edge_index[1].reshape(NW, NCHUNK, CH)

  hrel1, hroot1 = _tc_pre(x, W1_rel, W1_root, b1.reshape(1, -1))
  p = _segsum[16](hrel1, src, dst)
  h1 = _tc_combine1(p, hroot1)

  p = _segsum[16](h1, src, dst)
  h2 = _tc_combine(p, h1, W2_rel, W2_root, b2.reshape(1, -1))

  p = _segsum[32](h2, src, dst)
  h3 = _tc_combine(p, h2, W3_rel, W3_root, b3.reshape(1, -1))

  p = _segsum[64](h3, src, dst)
  out = _tc_final(p, h3, W4_rel, W4_root, b4.reshape(1, -1),
                  lin1_W, lin1_b.reshape(1, -1), batch.reshape(1, -1),
                  lin2_W, lin2_b.reshape(1, -1))
  return out.reshape(-1)


# speedup vs baseline: 10.5377x; 10.5377x over previous
import functools

import jax
import jax.numpy as jnp
from jax import lax
from jax.experimental import pallas as pl
from jax.experimental.pallas import tpu as pltpu
from jax.experimental.pallas import tpu_sc as plsc

N = 10000
E = 320000
G = 64
NC = 2
NS = 16
L = 16
NW = NC * NS
EPT = E // NW
CH = 80
NCHUNK = EPT // CH
NP = 10240
RPT = NP // NS

_f32 = jnp.float32


def _make_segsum(w):
  mesh = plsc.VectorSubcoreMesh(
      core_axis_name="c", subcore_axis_name="s", num_cores=NC, num_subcores=NS)

  @functools.partial(
      pl.kernel,
      out_type=jax.ShapeDtypeStruct((NC, NP, w), _f32),
      mesh=mesh,
      compiler_params=pltpu.CompilerParams(use_tc_tiling_on_sc=False),
      scratch_types=[
          pltpu.VMEM((NCHUNK, CH), jnp.int32),
          pltpu.VMEM((NCHUNK, CH), jnp.int32),
          pltpu.VMEM((CH, w), _f32),
          pltpu.VMEM((RPT, w), _f32),
          pltpu.VMEM_SHARED((NP, w), _f32),
          pltpu.SemaphoreType.DMA,
      ],
  )
  def segsum(table_hbm, src_hbm, dst_hbm, out_hbm,
             src_v, dst_v, rows_v, zero_v, acc_sh, sem):
    cid = lax.axis_index("c")
    sid = lax.axis_index("s")
    wid = cid * NS + sid

    pltpu.sync_copy(src_hbm.at[wid], src_v)
    pltpu.sync_copy(dst_hbm.at[wid], dst_v)

    def zbody(r, carry):
      for c in range(w // L):
        zero_v[r, pl.ds(c * L, L)] = jnp.zeros((L,), _f32)
      return carry
    lax.fori_loop(0, RPT, zbody, 0)
    pltpu.sync_copy(zero_v, acc_sh.at[pl.ds(sid * RPT, RPT)])
    plsc.subcore_barrier()

    def body(j, carry):
      pltpu.async_copy(table_hbm.at[src_v.at[j]], rows_v, sem).wait()
      pltpu.sync_copy(rows_v, acc_sh.at[dst_v.at[j]], add=True)
      return carry
    lax.fori_loop(0, NCHUNK, body, 0)
    plsc.subcore_barrier()

    pltpu.sync_copy(acc_sh.at[pl.ds(sid * RPT, RPT)],
                    out_hbm.at[cid, pl.ds(sid * RPT, RPT)])

  return segsum


_segsum = {w: _make_segsum(w) for w in (16, 32, 64)}


def _relu(x):
  return jnp.maximum(x, 0.0)


def _tc_pre_body(x_ref, wr_ref, wo_ref, b_ref, hrel_ref, hroot_ref):
  x = x_ref[...]
  hrel_ref[...] = jnp.dot(x, wr_ref[...], preferred_element_type=_f32)
  hroot_ref[...] = jnp.dot(x, wo_ref[...], preferred_element_type=_f32) + b_ref[...]


def _tc_combine1_body(p_ref, hroot_ref, h_ref):
  p = p_ref[...]
  h_ref[...] = _relu(p[0, :N] + p[1, :N] + hroot_ref[...])


def _tc_combine_body(p_ref, h_ref, wr_ref, wo_ref, b_ref, out_ref):
  p = p_ref[...]
  agg = p[0, :N] + p[1, :N]
  out_ref[...] = _relu(
      jnp.dot(agg, wr_ref[...], preferred_element_type=_f32)
      + jnp.dot(h_ref[...], wo_ref[...], preferred_element_type=_f32)
      + b_ref[...])


def _tc_final_body(p_ref, h_ref, wr_ref, wo_ref, b_ref,
                   l1w_ref, l1b_ref, batch_ref, l2w_ref, l2b_ref, out_ref):
  p = p_ref[...]
  agg = p[0, :N] + p[1, :N]
  h4 = _relu(
      jnp.dot(agg, wr_ref[...], preferred_element_type=_f32)
      + jnp.dot(h_ref[...], wo_ref[...], preferred_element_type=_f32)
      + b_ref[...])
  hl = jnp.dot(h4, l1w_ref[...], preferred_element_type=_f32) + l1b_ref[...]
  gid = lax.broadcasted_iota(jnp.int32, (G, N), 0)
  mask = (gid == batch_ref[...]).astype(_f32)
  sums = jnp.dot(mask, hl, preferred_element_type=_f32)
  counts = jnp.sum(mask, axis=1, keepdims=True)
  pooled = sums / jnp.maximum(counts, 1.0)
  out_ref[...] = jnp.dot(pooled, l2w_ref[...], preferred_element_type=_f32) + l2b_ref[...]


def _sds(shape):
  return jax.ShapeDtypeStruct(shape, _f32)


_tc_pre = pl.pallas_call(
    _tc_pre_body, out_shape=(_sds((N, 16)), _sds((N, 16))))

_tc_combine1 = pl.pallas_call(
    _tc_combine1_body, out_shape=_sds((N, 16)))


def _tc_combine(p, h, wr, wo, b):
  dout = wr.shape[1]
  return pl.pallas_call(_tc_combine_body, out_shape=_sds((N, dout)))(
      p, h, wr, wo, b)


_tc_final = pl.pallas_call(_tc_final_body, out_shape=_sds((G, 1)))


def kernel(x, edge_index, batch, W1_rel, W1_root, b1, W2_rel, W2_root, b2,
           W3_rel, W3_root, b3, W4_rel, W4_root, b4, lin1_W, lin1_b,
           lin2_W, lin2_b):
  src = edge_index[0].reshape(NW, NCHUNK, CH)
  dst = edge_index[1].reshape(NW, NCHUNK, CH)

  hrel1, hroot1 = _tc_pre(x, W1_rel, W1_root, b1.reshape(1, -1))
  p = _segsum[16](hrel1, src, dst)
  h1 = _tc_combine1(p, hroot1)

  p = _segsum[16](h1, src, dst)
  h2 = _tc_combine(p, h1, W2_rel, W2_root, b2.reshape(1, -1))

  p = _segsum[32](h2, src, dst)
  h3 = _tc_combine(p, h2, W3_rel, W3_root, b3.reshape(1, -1))

  p = _segsum[64](h3, src, dst)
  out = _tc_final(p, h3, W4_rel, W4_root, b4.reshape(1, -1),
                  lin1_W, lin1_b.reshape(1, -1), batch.reshape(1, -1),
                  lin2_W, lin2_b.reshape(1, -1))
  return out.reshape(-1)


# speedup vs baseline: 23.5929x; 2.2389x over previous
import functools

import jax
import jax.numpy as jnp
from jax import lax
from jax.experimental import pallas as pl
from jax.experimental.pallas import tpu as pltpu
from jax.experimental.pallas import tpu_sc as plsc

N = 10000
E = 320000
G = 64
NC = 2
NS = 16
L = 16
NW = NC * NS
EPT = E // NW
CH = 128
PAD = 240
EPTP = EPT + PAD
NCHUNK = EPTP // CH
NBUF = 4
NP = 10240
RPT = NP // NS
ZR = 128

_f32 = jnp.float32


def _make_segsum(w):
  mesh = plsc.VectorSubcoreMesh(
      core_axis_name="c", subcore_axis_name="s", num_cores=NC, num_subcores=NS)

  @functools.partial(
      pl.kernel,
      out_type=jax.ShapeDtypeStruct((NC, NP, w), _f32),
      mesh=mesh,
      compiler_params=pltpu.CompilerParams(use_tc_tiling_on_sc=False),
      scratch_types=[
          pltpu.VMEM((NCHUNK, CH), jnp.int32),
          pltpu.VMEM((NCHUNK, CH), jnp.int32),
          [pltpu.VMEM((CH, w), _f32)] * NBUF,
          pltpu.VMEM((ZR, w), _f32),
          pltpu.VMEM_SHARED((NP, w), _f32),
          [pltpu.SemaphoreType.DMA] * NBUF,
      ],
  )
  def segsum(table_hbm, src_hbm, dst_hbm, out_hbm,
             src_v, dst_v, rows_v, zero_v, acc_sh, sem):
    cid = lax.axis_index("c")
    sid = lax.axis_index("s")
    wid = cid * NS + sid

    pltpu.sync_copy(src_hbm.at[wid], src_v)
    pltpu.sync_copy(dst_hbm.at[wid], dst_v)

    def zbody(r, carry):
      for c in range(w // L):
        zero_v[r, pl.ds(c * L, L)] = jnp.zeros((L,), _f32)
      return carry
    lax.fori_loop(0, ZR, zbody, 0)
    for k in range(RPT // ZR):
      pltpu.sync_copy(zero_v, acc_sh.at[pl.ds(sid * RPT + k * ZR, ZR)])
    plsc.subcore_barrier()

    for b in range(NBUF):
      pltpu.async_copy(table_hbm.at[src_v.at[b]], rows_v[b], sem[b])

    def body(t, carry):
      for b in range(NBUF):
        j = t * NBUF + b
        pltpu.make_async_copy(
            table_hbm.at[src_v.at[j]], rows_v[b], sem[b]).wait()
        pltpu.sync_copy(rows_v[b], acc_sh.at[dst_v.at[j]], add=True)

        @pl.when(j + NBUF < NCHUNK)
        def _prefetch():
          pltpu.async_copy(
              table_hbm.at[src_v.at[j + NBUF]], rows_v[b], sem[b])
      return carry
    lax.fori_loop(0, NCHUNK // NBUF, body, 0)
    plsc.subcore_barrier()

    pltpu.sync_copy(acc_sh.at[pl.ds(sid * RPT, RPT)],
                    out_hbm.at[cid, pl.ds(sid * RPT, RPT)])

  return segsum


_segsum = {w: _make_segsum(w) for w in (16, 32, 64)}


def _relu(x):
  return jnp.maximum(x, 0.0)


def _tc_pre_body(x_ref, wr_ref, wo_ref, b_ref, hrel_ref, hroot_ref):
  x = x_ref[...]
  hrel_ref[...] = jnp.dot(x, wr_ref[...], preferred_element_type=_f32)
  hroot_ref[...] = jnp.dot(x, wo_ref[...], preferred_element_type=_f32) + b_ref[...]


def _tc_combine1_body(p_ref, hroot_ref, h_ref):
  p = p_ref[...]
  h_ref[...] = _relu(p[0, :N] + p[1, :N] + hroot_ref[...])


def _tc_combine_body(p_ref, h_ref, wr_ref, wo_ref, b_ref, out_ref):
  p = p_ref[...]
  agg = p[0, :N] + p[1, :N]
  out_ref[...] = _relu(
      jnp.dot(agg, wr_ref[...], preferred_element_type=_f32)
      + jnp.dot(h_ref[...], wo_ref[...], preferred_element_type=_f32)
      + b_ref[...])


def _tc_final_body(p_ref, h_ref, wr_ref, wo_ref, b_ref,
                   l1w_ref, l1b_ref, batch_ref, l2w_ref, l2b_ref, out_ref):
  p = p_ref[...]
  agg = p[0, :N] + p[1, :N]
  h4 = _relu(
      jnp.dot(agg, wr_ref[...], preferred_element_type=_f32)
      + jnp.dot(h_ref[...], wo_ref[...], preferred_element_type=_f32)
      + b_ref[...])
  hl = jnp.dot(h4, l1w_ref[...], preferred_element_type=_f32) + l1b_ref[...]
  gid = lax.broadcasted_iota(jnp.int32, (G, N), 0)
  mask = (gid == batch_ref[...]).astype(_f32)
  sums = jnp.dot(mask, hl, preferred_element_type=_f32)
  counts = jnp.sum(mask, axis=1, keepdims=True)
  pooled = sums / jnp.maximum(counts, 1.0)
  out_ref[...] = jnp.dot(pooled, l2w_ref[...], preferred_element_type=_f32) + l2b_ref[...]


def _sds(shape):
  return jax.ShapeDtypeStruct(shape, _f32)


_tc_pre = pl.pallas_call(
    _tc_pre_body, out_shape=(_sds((N, 16)), _sds((N, 16))))

_tc_combine1 = pl.pallas_call(
    _tc_combine1_body, out_shape=_sds((N, 16)))


def _tc_combine(p, h, wr, wo, b):
  dout = wr.shape[1]
  return pl.pallas_call(_tc_combine_body, out_shape=_sds((N, dout)))(
      p, h, wr, wo, b)


_tc_final = pl.pallas_call(_tc_final_body, out_shape=_sds((G, 1)))


def kernel(x, edge_index, batch, W1_rel, W1_root, b1, W2_rel, W2_root, b2,
           W3_rel, W3_root, b3, W4_rel, W4_root, b4, lin1_W, lin1_b,
           lin2_W, lin2_b):
  pad_src = jnp.broadcast_to((jnp.arange(PAD, dtype=jnp.int32) * 37) % N,
                             (NW, PAD))
  pad_dst = jnp.broadcast_to(N + jnp.arange(PAD, dtype=jnp.int32), (NW, PAD))
  src = jnp.concatenate(
      [edge_index[0].reshape(NW, EPT), pad_src], axis=1).reshape(NW, NCHUNK, CH)
  dst = jnp.concatenate(
      [edge_index[1].reshape(NW, EPT), pad_dst], axis=1).reshape(NW, NCHUNK, CH)

  hrel1, hroot1 = _tc_pre(x, W1_rel, W1_root, b1.reshape(1, -1))
  p = _segsum[16](hrel1, src, dst)
  h1 = _tc_combine1(p, hroot1)

  p = _segsum[16](h1, src, dst)
  h2 = _tc_combine(p, h1, W2_rel, W2_root, b2.reshape(1, -1))

  p = _segsum[32](h2, src, dst)
  h3 = _tc_combine(p, h2, W3_rel, W3_root, b3.reshape(1, -1))

  p = _segsum[64](h3, src, dst)
  out = _tc_final(p, h3, W4_rel, W4_root, b4.reshape(1, -1),
                  lin1_W, lin1_b.reshape(1, -1), batch.reshape(1, -1),
                  lin2_W, lin2_b.reshape(1, -1))
  return out.reshape(-1)
